# baseline (speedup 1.0000x reference)
import functools

import jax
import jax.numpy as jnp
from jax import lax
from jax.experimental import pallas as pl
from jax.experimental.pallas import tpu as pltpu
from jax.experimental.pallas import tpu_sc as plsc

NC, NS, L = 2, 16, 16
NW = NC * NS


def _make_sc_kernel(B, P, F, C):
    rows_per_w = B // NW
    nchunk = rows_per_w // C
    nvreg = P // L

    mesh = plsc.VectorSubcoreMesh(core_axis_name="c", subcore_axis_name="s")

    @functools.partial(
        pl.kernel,
        out_type=jax.ShapeDtypeStruct((B, P), jnp.float32),
        mesh=mesh,
        compiler_params=pltpu.CompilerParams(
            needs_layout_passes=False, use_tc_tiling_on_sc=True
        ),
        scratch_types=[
            pltpu.VMEM((F, C), jnp.float32),
            pltpu.VMEM((F, C), jnp.float32),
            pltpu.VMEM((C, P), jnp.float32),
            pltpu.VMEM((C, P), jnp.float32),
            pltpu.VMEM((P,), jnp.int32),
            pltpu.VMEM((P,), jnp.int32),
            pltpu.VMEM((P,), jnp.float32),
            pltpu.SemaphoreType.DMA,
            pltpu.SemaphoreType.DMA,
            pltpu.SemaphoreType.DMA,
            pltpu.SemaphoreType.DMA,
        ],
    )
    def sc_expand(params_hbm, gidx_hbm, free_hbm, dflt_hbm, out_hbm,
                  in0, in1, ob0, ob1, g_v, f_v, d_v, si0, si1, so0, so1):
        wid = lax.axis_index("s") * NC + lax.axis_index("c")
        row0 = wid * rows_per_w
        ins, outs, sis, sos = [in0, in1], [ob0, ob1], [si0, si1], [so0, so1]

        pltpu.sync_copy(gidx_hbm, g_v)
        pltpu.sync_copy(free_hbm, f_v)
        pltpu.sync_copy(dflt_hbm, d_v)

        gc = [g_v[pl.ds(L * v, L)] for v in range(nvreg)]
        dv = [d_v[pl.ds(L * v, L)] for v in range(nvreg)]
        mv = [f_v[pl.ds(L * v, L)] != 0 for v in range(nvreg)]

        def in_src(c):
            return params_hbm.at[:, pl.ds(row0 + c * C, C)]

        def out_dst(c):
            return out_hbm.at[pl.ds(row0 + c * C, C)]

        def in_dst(b):
            return ins[b]

        pltpu.async_copy(in_src(0), in_dst(0), sis[0])
        pltpu.async_copy(in_src(1), in_dst(1), sis[1])

        @pl.loop(0, nchunk, step=2)
        def _chunkpair(c0):
            for b in range(2):
                c = c0 + b
                pltpu.make_async_copy(in_src(c), in_dst(b), sis[b]).wait()

                @pl.when(c >= 2)
                def _():
                    pltpu.make_async_copy(outs[b], out_dst(c), sos[b]).wait()

                @plsc.parallel_loop(0, C, unroll=8)
                def _row(r):
                    rsplat = jnp.full((L,), r, dtype=jnp.int32)
                    for v in range(nvreg):
                        vals = plsc.load_gather(ins[b], [gc[v], rsplat])
                        outs[b][r, pl.ds(L * v, L)] = jnp.where(
                            mv[v], vals, dv[v]
                        )

                pltpu.async_copy(outs[b], out_dst(c), sos[b])

                @pl.when(c + 2 < nchunk)
                def _():
                    pltpu.async_copy(in_src(c + 2), in_dst(b), sis[b])

        pltpu.make_async_copy(outs[0], out_dst(nchunk - 2), sos[0]).wait()
        pltpu.make_async_copy(outs[1], out_dst(nchunk - 1), sos[1]).wait()

    return sc_expand


def kernel(params, params_default, free_inds):
    B, F = params.shape
    P = params_default.shape[0]
    is_free = jnp.zeros((P,), jnp.bool_).at[free_inds].set(True)
    inv = jnp.zeros((P,), jnp.int32).at[free_inds].set(
        jnp.arange(F, dtype=jnp.int32)
    )
    freec = is_free.astype(jnp.int32)
    excl = jnp.cumsum(freec) - freec
    vstart = (jnp.arange(P, dtype=jnp.int32) // L) * L
    off_v = excl[vstart]
    nf_v = excl[vstart + L - 1] + freec[vstart + L - 1] - off_v
    fixedc = 1 - freec
    frank = (jnp.cumsum(fixedc) - fixedc) - (vstart - off_v)
    gidx = jnp.where(is_free, inv, (off_v + nf_v + frank) % F).astype(jnp.int32)
    fn = _make_sc_kernel(B, P, F, C=128)
    return fn(
        params.T,
        gidx,
        freec,
        params_default.astype(jnp.float32),
    )


# speedup vs baseline: 1.8697x; 1.8697x over previous
import functools

import jax
import jax.numpy as jnp
from jax import lax
from jax.experimental import pallas as pl
from jax.experimental.pallas import tpu as pltpu
from jax.experimental.pallas import tpu_sc as plsc

NC, NS, L = 2, 16, 16
NW = NC * NS


def _make_sc_kernel(B, P, F, C):
    rows_per_w = B // NW
    nchunk = rows_per_w // C
    nvreg = P // L

    mesh = plsc.VectorSubcoreMesh(core_axis_name="c", subcore_axis_name="s")

    @functools.partial(
        pl.kernel,
        out_type=jax.ShapeDtypeStruct((B, P), jnp.float32),
        mesh=mesh,
        compiler_params=pltpu.CompilerParams(
            needs_layout_passes=False, use_tc_tiling_on_sc=True
        ),
        scratch_types=[
            pltpu.VMEM((C, F), jnp.float32),
            pltpu.VMEM((C, F), jnp.float32),
            pltpu.VMEM((C, P), jnp.float32),
            pltpu.VMEM((C, P), jnp.float32),
            pltpu.VMEM((P,), jnp.int32),
            pltpu.VMEM((P,), jnp.int32),
            pltpu.VMEM((P,), jnp.float32),
            pltpu.SemaphoreType.DMA,
            pltpu.SemaphoreType.DMA,
            pltpu.SemaphoreType.DMA,
            pltpu.SemaphoreType.DMA,
        ],
    )
    def sc_expand(params_hbm, gidx_hbm, free_hbm, dflt_hbm, out_hbm,
                  in0, in1, ob0, ob1, g_v, f_v, d_v, si0, si1, so0, so1):
        wid = lax.axis_index("s") * NC + lax.axis_index("c")
        row0 = wid * rows_per_w
        ins, outs, sis, sos = [in0, in1], [ob0, ob1], [si0, si1], [so0, so1]

        pltpu.sync_copy(gidx_hbm, g_v)
        pltpu.sync_copy(free_hbm, f_v)
        pltpu.sync_copy(dflt_hbm, d_v)

        gc = [g_v[pl.ds(L * v, L)] for v in range(nvreg)]
        dv = [d_v[pl.ds(L * v, L)] for v in range(nvreg)]
        mv = [f_v[pl.ds(L * v, L)] != 0 for v in range(nvreg)]

        def in_src(c):
            return params_hbm.at[pl.ds(row0 + c * C, C)]

        def out_dst(c):
            return out_hbm.at[pl.ds(row0 + c * C, C)]

        def in_dst(b):
            return ins[b]

        pltpu.async_copy(in_src(0), in_dst(0), sis[0])
        pltpu.async_copy(in_src(1), in_dst(1), sis[1])

        @pl.loop(0, nchunk, step=2)
        def _chunkpair(c0):
            for b in range(2):
                c = c0 + b
                pltpu.make_async_copy(in_src(c), in_dst(b), sis[b]).wait()

                @pl.when(c >= 2)
                def _():
                    pltpu.make_async_copy(outs[b], out_dst(c), sos[b]).wait()

                @plsc.parallel_loop(0, C, unroll=8)
                def _row(r):
                    rsplat = jnp.full((L,), r, dtype=jnp.int32)
                    for v in range(nvreg):
                        vals = plsc.load_gather(ins[b], [rsplat, gc[v]])
                        outs[b][r, pl.ds(L * v, L)] = jnp.where(
                            mv[v], vals, dv[v]
                        )

                pltpu.async_copy(outs[b], out_dst(c), sos[b])

                @pl.when(c + 2 < nchunk)
                def _():
                    pltpu.async_copy(in_src(c + 2), in_dst(b), sis[b])

        pltpu.make_async_copy(outs[0], out_dst(nchunk - 2), sos[0]).wait()
        pltpu.make_async_copy(outs[1], out_dst(nchunk - 1), sos[1]).wait()

    return sc_expand


def kernel(params, params_default, free_inds):
    B, F = params.shape
    P = params_default.shape[0]
    is_free = jnp.zeros((P,), jnp.bool_).at[free_inds].set(True)
    inv = jnp.zeros((P,), jnp.int32).at[free_inds].set(
        jnp.arange(F, dtype=jnp.int32)
    )
    freec = is_free.astype(jnp.int32)
    excl = jnp.cumsum(freec) - freec
    vstart = (jnp.arange(P, dtype=jnp.int32) // L) * L
    off_v = excl[vstart]
    nf_v = excl[vstart + L - 1] + freec[vstart + L - 1] - off_v
    fixedc = 1 - freec
    frank = (jnp.cumsum(fixedc) - fixedc) - (vstart - off_v)
    gidx = jnp.where(is_free, inv, (off_v + nf_v + frank) % F).astype(jnp.int32)
    fn = _make_sc_kernel(B, P, F, C=128)
    return fn(
        params,
        gidx,
        freec,
        params_default.astype(jnp.float32),
    )
